# baseline (speedup 1.0000x reference)
import functools

import numpy as np
import jax
import jax.numpy as jnp
from jax import lax
from jax.experimental import pallas as pl
from jax.experimental.pallas import tpu as pltpu
from jax.experimental.pallas import tpu_sc as plsc

SRC_DIMS = [32, 32, 32, 32]
SIZE = sum(SRC_DIMS)
SEQ = 50
BATCH = 4096
N = BATCH * SEQ

_info = plsc.get_sparse_core_info()
NC, NS, L = _info.num_cores, _info.num_subcores, _info.num_lanes
NW = NC * NS
PER_W = N // NW
ROWS_PER_CHUNK = 4
CHUNK = ROWS_PER_CHUNK * SEQ
N_CHUNKS = PER_W // CHUNK
NVEC = SIZE // L
G1 = 128
G2 = CHUNK - G1


def _positional_encoding_np(embedding_size, sequence_length):
    pe = np.zeros((sequence_length, embedding_size), dtype=np.float32)
    position = np.arange(sequence_length, dtype=np.float32)[:, None]
    div_term = np.exp(
        np.arange(0, embedding_size, 2, dtype=np.float32)
        * (-np.log(10000.0) / embedding_size))
    pe[:, 0::2] = np.sin(position * div_term)
    pe[:, 1::2] = np.cos(position * div_term)
    return pe


_PE = _positional_encoding_np(SIZE, SEQ)

_mesh = plsc.VectorSubcoreMesh(core_axis_name="c", subcore_axis_name="s")


@functools.partial(
    pl.kernel,
    out_type=jax.ShapeDtypeStruct((N, SIZE), jnp.float32),
    mesh=_mesh,
    scratch_types=[
        pltpu.VMEM((CHUNK + L,), jnp.float32),
        pltpu.VMEM((CHUNK + L,), jnp.float32),
        pltpu.VMEM((CHUNK + L,), jnp.float32),
        pltpu.VMEM((CHUNK,), jnp.int32),
        pltpu.VMEM((CHUNK, 32), jnp.float32),
        pltpu.VMEM((CHUNK, SIZE), jnp.float32),
        pltpu.VMEM((SEQ, SIZE), jnp.float32),
        pltpu.VMEM((SIZE,), jnp.float32),
        pltpu.SemaphoreType.DMA,
    ],
    compiler_params=pltpu.CompilerParams(use_tc_tiling_on_sc=False),
)
def _sc_embed(x0_hbm, x1_hbm, x2_hbm, idx_hbm, emb_hbm, add_hbm, w_hbm,
              out_hbm, x0_v, x1_v, x2_v, idx_v, rows_v, out_v, add_v, w_v,
              sem):
    wid = lax.axis_index("s") * NC + lax.axis_index("c")
    pltpu.sync_copy(add_hbm, add_v)
    pltpu.sync_copy(w_hbm, w_v)
    w_regs = [w_v[pl.ds(L * j, L)] for j in range(6)]
    x_hbms = [x0_hbm, x1_hbm, x2_hbm]
    x_vs = [x0_v, x1_v, x2_v]

    def chunk_body(k, _):
        base = wid * PER_W + k * CHUNK
        for i in range(3):
            pltpu.sync_copy(x_hbms[i].at[pl.ds(base, CHUNK)],
                            x_vs[i].at[pl.ds(0, CHUNK)])
        pltpu.sync_copy(idx_hbm.at[pl.ds(base, CHUNK)], idx_v)
        cp1 = pltpu.async_copy(emb_hbm.at[idx_v.at[pl.ds(0, G1)]],
                               rows_v.at[pl.ds(0, G1)], sem)
        cp2 = pltpu.async_copy(emb_hbm.at[idx_v.at[pl.ds(G1, G2)]],
                               rows_v.at[pl.ds(G1, G2)], sem)
        cp1.wait()
        cp2.wait()

        def s_body(s, _):
            adds = [add_v[s, pl.ds(L * j, L)] for j in range(NVEC)]
            for r in range(ROWS_PER_CHUNK):
                q = r * SEQ + s
                xs = [x_vs[i][pl.ds(q, L)][0] for i in range(3)]
                for j in range(NVEC):
                    if j < 6:
                        val = xs[j // 2] * w_regs[j] + adds[j]
                    else:
                        val = rows_v[q, pl.ds(L * (j - 6), L)] + adds[j]
                    out_v[q, pl.ds(L * j, L)] = val
            return 0

        lax.fori_loop(0, SEQ, s_body, 0)
        pltpu.sync_copy(out_v, out_hbm.at[pl.ds(base, CHUNK)])
        return 0

    lax.fori_loop(0, N_CHUNKS, chunk_body, 0)


def kernel(input_tensor, W0, b0, W1, b1, W2, b2, emb_table):
    B, S, _ = input_tensor.shape
    xcols = [input_tensor[:, :, i].reshape(N) for i in range(3)]
    idx32 = input_tensor[:, :, 3].astype(jnp.int32).reshape(N)
    bias = jnp.concatenate([b0, b1, b2, jnp.zeros((32,), jnp.float32)])
    add_tab = bias[None, :] + jnp.asarray(_PE)
    wvec = jnp.concatenate(
        [W0[0], W1[0], W2[0], jnp.ones((32,), jnp.float32)])
    out = _sc_embed(xcols[0], xcols[1], xcols[2], idx32, emb_table,
                    add_tab, wvec)
    return out.reshape(B, S, SIZE)
